# baseline (speedup 1.0000x reference)
import functools

import jax
import jax.numpy as jnp
from jax import lax
from jax.experimental import pallas as pl
from jax.experimental.pallas import tpu as pltpu
from jax.experimental.pallas import tpu_sc as plsc

G = 2
W = 10000
S = 50000
D = 128
E = 300000
N = W + S

NT = 16
EP = 311296
EB = EP // 128
TB = EB // NT
DC = 32
NDC = D // DC
GW_ROWS = 10240
GS_ROWS = 50176
WP = 10240
SP = 50176
ACC_ROWS = 60288
DEG_ROWS = 51200
IB = 4
BR = 1024

_SC_PARAMS = pltpu.CompilerParams(use_tc_tiling_on_sc=False)


def _sc_mesh():
    return plsc.VectorSubcoreMesh(core_axis_name="c", subcore_axis_name="s")


def _degrees(rows2d, cols2d):
    out_type = (
        jax.ShapeDtypeStruct((G, WP, 16), jnp.float32),
        jax.ShapeDtypeStruct((G, SP, 16), jnp.float32),
    )
    scratch = [
        pltpu.VMEM((TB, 128), jnp.int32),
        pltpu.VMEM((128, 16), jnp.float32),
        pltpu.VMEM((128, 16), jnp.float32),
        pltpu.VMEM_SHARED((DEG_ROWS, 16), jnp.float32),
    ]

    @functools.partial(pl.kernel, out_type=out_type, mesh=_sc_mesh(),
                       scratch_types=scratch, compiler_params=_SC_PARAMS)
    def deg_kernel(rows_hbm, cols_hbm, degw_hbm, degs_hbm,
                   idx_v, basis, zb, acc):
        cid = lax.axis_index("c")
        sid = lax.axis_index("s")
        lane = lax.iota(jnp.int32, 16)
        one16 = jnp.where(lane == 0, 1.0, 0.0).astype(jnp.float32)
        zero16 = jnp.zeros((16,), jnp.float32)

        @pl.loop(0, 128)
        def _(i):
            basis[i, :] = one16
            zb[i, :] = zero16

        for g in range(G):
            @pl.loop(0, DEG_ROWS // NT // 128)
            def _(i):
                pltpu.sync_copy(zb, acc.at[pl.ds(sid * (DEG_ROWS // NT)
                                                 + i * 128, 128)])
            plsc.subcore_barrier()

            @pl.when(cid == 0)
            def _():
                pltpu.sync_copy(rows_hbm.at[g, pl.ds(sid * TB, TB)], idx_v)

            @pl.when(cid == 1)
            def _():
                pltpu.sync_copy(cols_hbm.at[g, pl.ds(sid * TB, TB)], idx_v)

            @pl.loop(0, TB)
            def _(j):
                pltpu.sync_copy(basis, acc.at[idx_v.at[j]], add=True)

            plsc.subcore_barrier()

            @pl.when(cid == 0)
            def _():
                pltpu.sync_copy(acc.at[pl.ds(sid * (WP // NT), WP // NT)],
                                degw_hbm.at[g, pl.ds(sid * (WP // NT),
                                                     WP // NT)])

            @pl.when(cid == 1)
            def _():
                pltpu.sync_copy(acc.at[pl.ds(sid * (SP // NT), SP // NT)],
                                degs_hbm.at[g, pl.ds(sid * (SP // NT),
                                                     SP // NT)])
            plsc.subcore_barrier()

    return deg_kernel(rows2d, cols2d)


def _spmm(rows2d, cols2d, gw4, gs4):
    out_type = (
        jax.ShapeDtypeStruct((G, NDC, WP, DC), jnp.float32),
        jax.ShapeDtypeStruct((G, NDC, SP, DC), jnp.float32),
    )
    scratch = [
        pltpu.VMEM((2 * IB, 64), jnp.int32),
        pltpu.VMEM((2 * IB, 64), jnp.int32),
        pltpu.VMEM((2 * IB, 64), jnp.int32),
        pltpu.VMEM((4, 64, DC), jnp.float32),
        pltpu.VMEM_SHARED((ACC_ROWS, DC), jnp.float32),
        pltpu.SemaphoreType.DMA((4,)),
        pltpu.SemaphoreType.DMA((4,)),
    ]
    NB = 4 * IB

    rows2d = rows2d.reshape(G, 2 * EB, 64)
    cols2d = cols2d.reshape(G, 2 * EB, 64)

    @functools.partial(pl.kernel, out_type=out_type, mesh=_sc_mesh(),
                       scratch_types=scratch, compiler_params=_SC_PARAMS)
    def spmm_kernel(rows_hbm, cols_hbm, gw_hbm, gs_hbm,
                    raww_hbm, raws_hbm,
                    ridx, cidx, cidxP, stage, acc, sem_g, sem_s):
        cid = lax.axis_index("c")
        sid = lax.axis_index("s")
        zero16 = jnp.zeros((16,), jnp.float32)

        for g in range(G):
            for k in range(NDC // 2):
                dc = 2 * cid + k

                @pl.loop(0, 64)
                def _(i):
                    @pl.loop(0, DC, step=16)
                    def _(j):
                        stage[0, i, pl.ds(j, 16)] = zero16

                @pl.loop(sid, ACC_ROWS // 64, step=NT)
                def _(i):
                    pltpu.sync_copy(stage.at[0],
                                    acc.at[pl.ds(i * 64, 64)])
                plsc.subcore_barrier()

                def half_idx(buf, b):
                    return buf.at[b % (2 * IB)]

                def gather_src(b):
                    if b < NB // 2:
                        return gs_hbm.at[g, dc].at[half_idx(cidx, b)]
                    return gw_hbm.at[g, dc].at[half_idx(ridx, b - NB // 2)]

                def scatter_dst(b):
                    if b < NB // 2:
                        return acc.at[half_idx(ridx, b)]
                    return acc.at[half_idx(cidxP, b - NB // 2)]

                @pl.loop(0, TB // IB)
                def _(jo):
                    base = 2 * (sid * TB + jo * IB)
                    pltpu.sync_copy(rows_hbm.at[g, pl.ds(base, 2 * IB)],
                                    ridx)
                    pltpu.sync_copy(cols_hbm.at[g, pl.ds(base, 2 * IB)],
                                    cidx)

                    @pl.loop(0, 2 * IB)
                    def _(jj):
                        @pl.loop(0, 64, step=16)
                        def _(j):
                            cidxP[jj, pl.ds(j, 16)] = (
                                cidx[jj, pl.ds(j, 16)] + W)

                    gd = {}
                    sd = {}
                    for b in range(4):
                        gd[b] = pltpu.async_copy(gather_src(b),
                                                 stage.at[b],
                                                 sem_g.at[b])
                    for b in range(NB):
                        if b + 2 < NB and b + 2 >= 4:
                            sd[b - 2].wait()
                            gd[b + 2] = pltpu.async_copy(
                                gather_src(b + 2), stage.at[(b + 2) % 4],
                                sem_g.at[(b + 2) % 4])
                        gd[b].wait()
                        sd[b] = pltpu.async_copy(stage.at[b % 4],
                                                 scatter_dst(b),
                                                 sem_s.at[b % 4], add=True)
                    for b in range(max(0, NB - 4), NB):
                        sd[b].wait()

                plsc.subcore_barrier()

                pltpu.sync_copy(
                    acc.at[pl.ds(sid * (WP // NT), WP // NT)],
                    raww_hbm.at[g, dc, pl.ds(sid * (WP // NT), WP // NT)])
                pltpu.sync_copy(
                    acc.at[pl.ds(W + sid * (SP // NT), SP // NT)],
                    raws_hbm.at[g, dc, pl.ds(sid * (SP // NT), SP // NT)])
                plsc.subcore_barrier()

    return spmm_kernel(rows2d, cols2d, gw4, gs4)


def _tc_scale(feats, deg, rows_real, table_rows):
    nb = table_rows // BR

    def body(f_ref, d_ref, o_ref):
        i = pl.program_id(1)
        f = f_ref[0]
        deg0 = d_ref[0, :, 0:1]
        d = 1.0 / (jnp.sqrt(deg0) + 1e-8)
        rowid = i * BR + lax.broadcasted_iota(jnp.int32, (BR, 1), 0)
        gv = jnp.where(rowid < rows_real, f * d, 0.0)
        for k in range(NDC):
            o_ref[0, k] = gv[:, DC * k:DC * (k + 1)]

    return pl.pallas_call(
        body,
        grid=(G, nb),
        in_specs=[
            pl.BlockSpec((1, BR, D), lambda b, i: (b, i, 0)),
            pl.BlockSpec((1, BR, 16), lambda b, i: (b, i, 0)),
        ],
        out_specs=pl.BlockSpec((1, NDC, BR, DC), lambda b, i: (b, 0, i, 0)),
        out_shape=jax.ShapeDtypeStruct((G, NDC, table_rows, DC), jnp.float32),
    )(feats, deg)


def _tc_mid(raw4, deg, f0, rows_real, table_rows, divisor):
    nb = table_rows // BR

    def body(r_ref, d_ref, f_ref, o_g_ref, o_a_ref):
        i = pl.program_id(1)
        raw = jnp.concatenate([r_ref[0, k] for k in range(NDC)], axis=-1)
        deg0 = d_ref[0, :, 0:1]
        d = 1.0 / (jnp.sqrt(deg0) + 1e-8)
        f1 = d * raw * (1.0 / divisor)
        nrm = jnp.sqrt(jnp.sum(f1 * f1, axis=-1, keepdims=True))
        accp = f_ref[0] + f1 / jnp.maximum(nrm, 1e-12)
        rowid = i * BR + lax.broadcasted_iota(jnp.int32, (BR, 1), 0)
        g1 = jnp.where(rowid < rows_real, d * f1, 0.0)
        for k in range(NDC):
            o_g_ref[0, k] = g1[:, DC * k:DC * (k + 1)]
        o_a_ref[0] = accp

    return pl.pallas_call(
        body,
        grid=(G, nb),
        in_specs=[
            pl.BlockSpec((1, NDC, BR, DC), lambda b, i: (b, 0, i, 0)),
            pl.BlockSpec((1, BR, 16), lambda b, i: (b, i, 0)),
            pl.BlockSpec((1, BR, D), lambda b, i: (b, i, 0)),
        ],
        out_specs=[
            pl.BlockSpec((1, NDC, BR, DC), lambda b, i: (b, 0, i, 0)),
            pl.BlockSpec((1, BR, D), lambda b, i: (b, i, 0)),
        ],
        out_shape=[
            jax.ShapeDtypeStruct((G, NDC, table_rows, DC), jnp.float32),
            jax.ShapeDtypeStruct((G, rows_real, D), jnp.float32),
        ],
    )(raw4, deg, f0)


def _tc_final(raw4, deg, accp, rows_real, table_rows, divisor):
    nb = table_rows // BR

    def body(r_ref, d_ref, a_ref, o_ref):
        raw = jnp.concatenate([r_ref[0, k] for k in range(NDC)], axis=-1)
        deg0 = d_ref[0, :, 0:1]
        d = 1.0 / (jnp.sqrt(deg0) + 1e-8)
        f2 = d * raw * (1.0 / divisor)
        nrm = jnp.sqrt(jnp.sum(f2 * f2, axis=-1, keepdims=True))
        o_ref[0] = a_ref[0] + f2 / jnp.maximum(nrm, 1e-12)

    return pl.pallas_call(
        body,
        grid=(G, nb),
        in_specs=[
            pl.BlockSpec((1, NDC, BR, DC), lambda b, i: (b, 0, i, 0)),
            pl.BlockSpec((1, BR, 16), lambda b, i: (b, i, 0)),
            pl.BlockSpec((1, BR, D), lambda b, i: (b, i, 0)),
        ],
        out_specs=pl.BlockSpec((1, BR, D), lambda b, i: (b, i, 0)),
        out_shape=jax.ShapeDtypeStruct((G, rows_real, D), jnp.float32),
    )(raw4, deg, accp)


def _pad_idx(r, pad_val):
    return jnp.concatenate(
        [r, jnp.full((EP - E,), pad_val, jnp.int32)]).reshape(EB, 128)


def kernel(edge_rows_0, edge_cols_0, edge_rows_1, edge_cols_1,
           warehouse_features, site_features):
    rows2d = jnp.stack([_pad_idx(edge_rows_0, W), _pad_idx(edge_rows_1, W)])
    cols2d = jnp.stack([_pad_idx(edge_cols_0, S), _pad_idx(edge_cols_1, S)])

    degw, degs = _degrees(rows2d, cols2d)

    gw0 = _tc_scale(warehouse_features, degw, W, GW_ROWS)
    gs0 = _tc_scale(site_features, degs, S, GS_ROWS)

    raww1, raws1 = _spmm(rows2d, cols2d, gw0, gs0)

    gw1, accw = _tc_mid(raww1, degw, warehouse_features, W, GW_ROWS, 2.0)
    gs1, accs = _tc_mid(raws1, degs, site_features, S, GS_ROWS, 2.0)

    raww2, raws2 = _spmm(rows2d, cols2d, gw1, gs1)

    wf_out = _tc_final(raww2, degw, accw, W, GW_ROWS, 3.0)
    sf_out = _tc_final(raws2, degs, accs, S, GS_ROWS, 3.0)
    return wf_out, sf_out


# speedup vs baseline: 1.0924x; 1.0924x over previous
import functools

import jax
import jax.numpy as jnp
from jax import lax
from jax.experimental import pallas as pl
from jax.experimental.pallas import tpu as pltpu
from jax.experimental.pallas import tpu_sc as plsc

G = 2
W = 10000
S = 50000
D = 128
E = 300000
N = W + S

NT = 16
EP = 311296
EB = EP // 64
TB = EP // 128 // NT
DC = 32
NDC = D // DC
GW_ROWS = 10240
GS_ROWS = 50176
WP = 10240
SP = 50176
ACC_ROWS = 60288
DEG_ROWS = 51200
IB = 4
BR = 1024

_SC_PARAMS = pltpu.CompilerParams(use_tc_tiling_on_sc=False)


def _sc_mesh():
    return plsc.VectorSubcoreMesh(core_axis_name="c", subcore_axis_name="s")


def _degrees(rows2d, cols2d):
    out_type = (
        jax.ShapeDtypeStruct((G, WP, 16), jnp.float32),
        jax.ShapeDtypeStruct((G, SP, 16), jnp.float32),
    )
    scratch = [
        pltpu.VMEM((2 * TB, 64), jnp.int32),
        pltpu.VMEM((64, 16), jnp.float32),
        pltpu.VMEM((64, 16), jnp.float32),
        pltpu.VMEM_SHARED((DEG_ROWS, 16), jnp.float32),
    ]

    @functools.partial(pl.kernel, out_type=out_type, mesh=_sc_mesh(),
                       scratch_types=scratch, compiler_params=_SC_PARAMS)
    def deg_kernel(rows_hbm, cols_hbm, degw_hbm, degs_hbm,
                   idx_v, basis, zb, acc):
        cid = lax.axis_index("c")
        sid = lax.axis_index("s")
        lane = lax.iota(jnp.int32, 16)
        one16 = jnp.where(lane == 0, 1.0, 0.0).astype(jnp.float32)
        zero16 = jnp.zeros((16,), jnp.float32)

        @pl.loop(0, 64)
        def _(i):
            basis[i, :] = one16
            zb[i, :] = zero16

        for g in range(G):
            @pl.loop(0, DEG_ROWS // NT // 64)
            def _(i):
                pltpu.sync_copy(zb, acc.at[pl.ds(sid * (DEG_ROWS // NT)
                                                 + i * 64, 64)])
            plsc.subcore_barrier()

            @pl.when(cid == 0)
            def _():
                pltpu.sync_copy(rows_hbm.at[g, pl.ds(sid * 2 * TB, 2 * TB)],
                                idx_v)

            @pl.when(cid == 1)
            def _():
                pltpu.sync_copy(cols_hbm.at[g, pl.ds(sid * 2 * TB, 2 * TB)],
                                idx_v)

            @pl.loop(0, 2 * TB)
            def _(j):
                pltpu.sync_copy(basis, acc.at[idx_v.at[j]], add=True)

            plsc.subcore_barrier()

            @pl.when(cid == 0)
            def _():
                pltpu.sync_copy(acc.at[pl.ds(sid * (WP // NT), WP // NT)],
                                degw_hbm.at[g, pl.ds(sid * (WP // NT),
                                                     WP // NT)])

            @pl.when(cid == 1)
            def _():
                pltpu.sync_copy(acc.at[pl.ds(sid * (SP // NT), SP // NT)],
                                degs_hbm.at[g, pl.ds(sid * (SP // NT),
                                                     SP // NT)])
            plsc.subcore_barrier()

    return deg_kernel(rows2d, cols2d)


def _spmm(rows2d, cols2d, gw4, gs4):
    out_type = (
        jax.ShapeDtypeStruct((NDC, WP, DC), jnp.float32),
        jax.ShapeDtypeStruct((NDC, SP, DC), jnp.float32),
    )
    scratch = [
        pltpu.VMEM((2 * IB, 64), jnp.int32),
        pltpu.VMEM((2 * IB, 64), jnp.int32),
        pltpu.VMEM((2 * IB, 64), jnp.int32),
        pltpu.VMEM((4, 64, DC), jnp.float32),
        pltpu.VMEM_SHARED((ACC_ROWS, DC), jnp.float32),
        pltpu.SemaphoreType.DMA((4,)),
        pltpu.SemaphoreType.DMA((4,)),
    ]
    NB = 4 * IB

    @functools.partial(pl.kernel, out_type=out_type, mesh=_sc_mesh(),
                       scratch_types=scratch, compiler_params=_SC_PARAMS)
    def spmm_kernel(rows_hbm, cols_hbm, gw_hbm, gs_hbm,
                    raww_hbm, raws_hbm,
                    ridx, cidx, cidxP, stage, acc, sem_g, sem_s):
        cid = lax.axis_index("c")
        sid = lax.axis_index("s")
        zero16 = jnp.zeros((16,), jnp.float32)

        for k in range(NDC // 2):
            dc = 2 * cid + k

            @pl.loop(0, 64)
            def _(i):
                @pl.loop(0, DC, step=16)
                def _(j):
                    stage[0, i, pl.ds(j, 16)] = zero16

            @pl.loop(sid, ACC_ROWS // 64, step=NT)
            def _(i):
                pltpu.sync_copy(stage.at[0], acc.at[pl.ds(i * 64, 64)])
            plsc.subcore_barrier()

            def half_idx(buf, b):
                return buf.at[b % (2 * IB)]

            def gather_src(b):
                if b < NB // 2:
                    return gs_hbm.at[dc].at[half_idx(cidx, b)]
                return gw_hbm.at[dc].at[half_idx(ridx, b - NB // 2)]

            def scatter_dst(b):
                if b < NB // 2:
                    return acc.at[half_idx(ridx, b)]
                return acc.at[half_idx(cidxP, b - NB // 2)]

            @pl.loop(0, TB // IB)
            def _(jo):
                base = 2 * (sid * TB + jo * IB)
                pltpu.sync_copy(rows_hbm.at[pl.ds(base, 2 * IB)], ridx)
                pltpu.sync_copy(cols_hbm.at[pl.ds(base, 2 * IB)], cidx)

                @pl.loop(0, 2 * IB)
                def _(jj):
                    @pl.loop(0, 64, step=16)
                    def _(j):
                        cidxP[jj, pl.ds(j, 16)] = (
                            cidx[jj, pl.ds(j, 16)] + W)

                gd = {}
                sd = {}
                for b in range(4):
                    gd[b] = pltpu.async_copy(gather_src(b), stage.at[b],
                                             sem_g.at[b])
                for b in range(NB):
                    if b + 2 < NB and b + 2 >= 4:
                        sd[b - 2].wait()
                        gd[b + 2] = pltpu.async_copy(
                            gather_src(b + 2), stage.at[(b + 2) % 4],
                            sem_g.at[(b + 2) % 4])
                    gd[b].wait()
                    sd[b] = pltpu.async_copy(stage.at[b % 4],
                                             scatter_dst(b),
                                             sem_s.at[b % 4], add=True)
                for b in range(max(0, NB - 4), NB):
                    sd[b].wait()

            plsc.subcore_barrier()

            pltpu.sync_copy(
                acc.at[pl.ds(sid * (WP // NT), WP // NT)],
                raww_hbm.at[dc, pl.ds(sid * (WP // NT), WP // NT)])
            pltpu.sync_copy(
                acc.at[pl.ds(W + sid * (SP // NT), SP // NT)],
                raws_hbm.at[dc, pl.ds(sid * (SP // NT), SP // NT)])
            plsc.subcore_barrier()

    return spmm_kernel(rows2d, cols2d, gw4, gs4)


def _tc_scale(feats, deg, rows_real, table_rows):
    nb = table_rows // BR

    def body(f_ref, d_ref, o_ref):
        i = pl.program_id(0)
        f = f_ref[...]
        deg0 = d_ref[:, 0:1]
        d = 1.0 / (jnp.sqrt(deg0) + 1e-8)
        rowid = i * BR + lax.broadcasted_iota(jnp.int32, (BR, 1), 0)
        gv = jnp.where(rowid < rows_real, f * d, 0.0)
        for k in range(NDC):
            o_ref[k] = gv[:, DC * k:DC * (k + 1)]

    return pl.pallas_call(
        body,
        grid=(nb,),
        in_specs=[
            pl.BlockSpec((BR, D), lambda i: (i, 0)),
            pl.BlockSpec((BR, 16), lambda i: (i, 0)),
        ],
        out_specs=pl.BlockSpec((NDC, BR, DC), lambda i: (0, i, 0)),
        out_shape=jax.ShapeDtypeStruct((NDC, table_rows, DC), jnp.float32),
    )(feats, deg)


def _tc_mid(raw4, deg, f0, rows_real, table_rows, divisor):
    nb = table_rows // BR

    def body(r_ref, d_ref, f_ref, o_g_ref, o_a_ref):
        i = pl.program_id(0)
        raw = jnp.concatenate([r_ref[k] for k in range(NDC)], axis=-1)
        deg0 = d_ref[:, 0:1]
        d = 1.0 / (jnp.sqrt(deg0) + 1e-8)
        f1 = d * raw * (1.0 / divisor)
        nrm = jnp.sqrt(jnp.sum(f1 * f1, axis=-1, keepdims=True))
        accp = f_ref[...] + f1 / jnp.maximum(nrm, 1e-12)
        rowid = i * BR + lax.broadcasted_iota(jnp.int32, (BR, 1), 0)
        g1 = jnp.where(rowid < rows_real, d * f1, 0.0)
        for k in range(NDC):
            o_g_ref[k] = g1[:, DC * k:DC * (k + 1)]
        o_a_ref[...] = accp

    return pl.pallas_call(
        body,
        grid=(nb,),
        in_specs=[
            pl.BlockSpec((NDC, BR, DC), lambda i: (0, i, 0)),
            pl.BlockSpec((BR, 16), lambda i: (i, 0)),
            pl.BlockSpec((BR, D), lambda i: (i, 0)),
        ],
        out_specs=[
            pl.BlockSpec((NDC, BR, DC), lambda i: (0, i, 0)),
            pl.BlockSpec((BR, D), lambda i: (i, 0)),
        ],
        out_shape=[
            jax.ShapeDtypeStruct((NDC, table_rows, DC), jnp.float32),
            jax.ShapeDtypeStruct((rows_real, D), jnp.float32),
        ],
    )(raw4, deg, f0)


def _tc_final(raw4, deg, accp, rows_real, table_rows, divisor):
    nb = table_rows // BR

    def body(r_ref, d_ref, a_ref, o_ref):
        raw = jnp.concatenate([r_ref[k] for k in range(NDC)], axis=-1)
        deg0 = d_ref[:, 0:1]
        d = 1.0 / (jnp.sqrt(deg0) + 1e-8)
        f2 = d * raw * (1.0 / divisor)
        nrm = jnp.sqrt(jnp.sum(f2 * f2, axis=-1, keepdims=True))
        o_ref[...] = a_ref[...] + f2 / jnp.maximum(nrm, 1e-12)

    return pl.pallas_call(
        body,
        grid=(nb,),
        in_specs=[
            pl.BlockSpec((NDC, BR, DC), lambda i: (0, i, 0)),
            pl.BlockSpec((BR, 16), lambda i: (i, 0)),
            pl.BlockSpec((BR, D), lambda i: (i, 0)),
        ],
        out_specs=pl.BlockSpec((BR, D), lambda i: (i, 0)),
        out_shape=jax.ShapeDtypeStruct((rows_real, D), jnp.float32),
    )(raw4, deg, accp)


def _pad_idx(r, pad_val):
    return jnp.concatenate(
        [r, jnp.full((EP - E,), pad_val, jnp.int32)]).reshape(EB, 64)


def kernel(edge_rows_0, edge_cols_0, edge_rows_1, edge_cols_1,
           warehouse_features, site_features):
    rows2 = [_pad_idx(edge_rows_0, W), _pad_idx(edge_rows_1, W)]
    cols2 = [_pad_idx(edge_cols_0, S), _pad_idx(edge_cols_1, S)]

    degw, degs = _degrees(jnp.stack(rows2), jnp.stack(cols2))

    wf_out = []
    sf_out = []
    for g in range(G):
        wf = warehouse_features[g]
        sf = site_features[g]
        dw = degw[g]
        ds = degs[g]
        gw0 = _tc_scale(wf, dw, W, GW_ROWS)
        gs0 = _tc_scale(sf, ds, S, GS_ROWS)
        raww1, raws1 = _spmm(rows2[g], cols2[g], gw0, gs0)
        gw1, accw = _tc_mid(raww1, dw, wf, W, GW_ROWS, 2.0)
        gs1, accs = _tc_mid(raws1, ds, sf, S, GS_ROWS, 2.0)
        raww2, raws2 = _spmm(rows2[g], cols2[g], gw1, gs1)
        wf_out.append(_tc_final(raww2, dw, accw, W, GW_ROWS, 3.0))
        sf_out.append(_tc_final(raws2, ds, accs, S, GS_ROWS, 3.0))
    return jnp.stack(wf_out), jnp.stack(sf_out)
